# baseline (speedup 1.0000x reference)
import dataclasses
import functools

import jax
import jax.numpy as jnp
from jax import lax
from jax.experimental import pallas as pl
from jax.experimental.pallas import tpu as pltpu
from jax.experimental.pallas import tpu_sc as plsc

V = 1000000
D = 64
B = 16384
C = 20
K = 17

NW = 32
PER_W = B // NW
CHUNK = 32
NCHUNK = PER_W // CHUNK
NB = 1


def _sc_scores_body(wcen, wctx, ctxT, cnF, out, ctxi, cni, scores,
                    accA, accB, rowsA, rowsB,
                    sem_cA, sem_cB, sem_rA, sem_rB):
    wid = lax.axis_index("s") * 2 + lax.axis_index("c")
    wbase = wid * PER_W

    pltpu.sync_copy(ctxT.at[:, pl.ds(wbase, PER_W)], ctxi)
    pltpu.sync_copy(cnF.at[pl.ds(wbase * K, PER_W * K)], cni)

    zeros16 = jnp.zeros((16,), jnp.float32)

    @pl.loop(0, K)
    def _(k):
        @pl.loop(0, PER_W, step=16)
        def _(i):
            scores[k, pl.ds(i, 16)] = zeros16

    def zero_acc(acc):
        @pl.loop(0, CHUNK)
        def _(i):
            for b in range(NB):
                for j in range(D // 16):
                    acc[b, i, pl.ds(j * 16, 16)] = zeros16

    def issue(n, acc, rows, sem_c, sem_r):
        for c in range(C):
            pltpu.async_copy(wctx.at[ctxi.at[c, pl.ds(n * CHUNK, CHUNK)]],
                             acc.at[c % NB], sem_c, add=True)
        pltpu.async_copy(wcen.at[cni.at[pl.ds(n * CHUNK * K, CHUNK * K)]],
                         rows, sem_r)

    def wait(acc, rows, sem_c, sem_r):
        for c in range(C):
            pltpu.make_async_copy(
                wctx.at[ctxi.at[0, pl.ds(0, CHUNK)]], acc.at[c % NB],
                sem_c).wait()
        pltpu.make_async_copy(
            wcen.at[cni.at[pl.ds(0, CHUNK * K)]], rows, sem_r).wait()

    def do_chunk(n, acc_cur, rows_cur, sem_c_cur, sem_r_cur,
                 acc_nxt, rows_nxt, sem_c_nxt, sem_r_nxt):
        @pl.when(n + 1 < NCHUNK)
        def _():
            zero_acc(acc_nxt)
            issue(n + 1, acc_nxt, rows_nxt, sem_c_nxt, sem_r_nxt)

        wait(acc_cur, rows_cur, sem_c_cur, sem_r_cur)

        base = n * CHUNK
        lane = lax.iota(jnp.int32, 16)
        last = lane == 15

        @pl.loop(0, CHUNK)
        def _(i):
            cv = [acc_cur[0, i, pl.ds(j * 16, 16)] for j in range(D // 16)]
            for b in range(1, NB):
                cv = [cv[j] + acc_cur[b, i, pl.ds(j * 16, 16)]
                      for j in range(D // 16)]
            col = jnp.full((16,), base + i, jnp.int32)
            for k in range(K):
                w = rows_cur[i * K + k, pl.ds(0, 16)] * cv[0]
                for j in range(1, D // 16):
                    w = w + rows_cur[i * K + k, pl.ds(j * 16, 16)] * cv[j]
                plsc.addupdate_scatter(scores,
                                       [jnp.full((16,), k, jnp.int32), col],
                                       w)

    zero_acc(accA)
    issue(0, accA, rowsA, sem_cA, sem_rA)

    @pl.loop(0, NCHUNK, step=2)
    def _(s):
        do_chunk(s, accA, rowsA, sem_cA, sem_rA, accB, rowsB, sem_cB, sem_rB)
        do_chunk(s + 1, accB, rowsB, sem_cB, sem_rB, accA, rowsA, sem_cA, sem_rA)

    pltpu.sync_copy(scores, out.at[:, pl.ds(wbase, PER_W)])


@jax.jit
def _sc_scores(wcen, wctx, ctxT, cnT):
    mesh = plsc.VectorSubcoreMesh(core_axis_name="c", subcore_axis_name="s")
    cp = pltpu.CompilerParams(use_tc_tiling_on_sc=False)
    if "needs_layout_passes" in pltpu.CompilerParams.__dataclass_fields__:
        cp = dataclasses.replace(cp, needs_layout_passes=False)
    f = pl.kernel(
        _sc_scores_body,
        out_type=jax.ShapeDtypeStruct((K, B), jnp.float32),
        mesh=mesh,
        scratch_types=[
            pltpu.VMEM((C, PER_W), jnp.int32),
            pltpu.VMEM((PER_W * K,), jnp.int32),
            pltpu.VMEM((K, PER_W), jnp.float32),
            pltpu.VMEM((NB, CHUNK, D), jnp.float32),
            pltpu.VMEM((NB, CHUNK, D), jnp.float32),
            pltpu.VMEM((CHUNK * K, D), jnp.float32),
            pltpu.VMEM((CHUNK * K, D), jnp.float32),
            pltpu.SemaphoreType.DMA,
            pltpu.SemaphoreType.DMA,
            pltpu.SemaphoreType.DMA,
            pltpu.SemaphoreType.DMA,
        ],
        compiler_params=cp,
    )
    return f(wcen, wctx, ctxT, cnT)


def _tc_loss_body(s_ref, o_ref):
    x = s_ref[...]
    zp = x[0:1, :]
    zn = -x[1:K, :]
    lsp = jnp.minimum(zp, 0.0) - jnp.log1p(jnp.exp(-jnp.abs(zp)))
    lsn = jnp.minimum(zn, 0.0) - jnp.log1p(jnp.exp(-jnp.abs(zn)))
    o_ref[0, 0] = -(jnp.sum(lsp) + jnp.sum(lsn)) / float(K * B)


@jax.jit
def _tc_loss(scores):
    return pl.pallas_call(
        _tc_loss_body,
        out_shape=jax.ShapeDtypeStruct((1, 1), jnp.float32),
        in_specs=[pl.BlockSpec((K, B), lambda: (0, 0))],
        out_specs=pl.BlockSpec(memory_space=pltpu.SMEM),
    )(scores)


RBH = 8192
NBLK = 61
VPAD = NBLK * RBH
VTAIL = 2 * VPAD
NROW = V // 2


def _tpose_body(lo_ref, hi_ref, tail_ref, out_ref):
    i = pl.program_id(0)
    eye = (lax.broadcasted_iota(jnp.int32, (2 * D, 2 * D), 0) ==
           lax.broadcasted_iota(jnp.int32, (2 * D, 2 * D), 1)).astype(jnp.float32)

    @pl.when(i < NBLK)
    def _():
        x = jnp.concatenate([lo_ref[...], hi_ref[...]], axis=0)
        out_ref[...] = lax.dot_general(x, eye, (((0,), (0,)), ((), ())),
                                       preferred_element_type=jnp.float32)

    @pl.when(i == NBLK)
    def _():
        out_ref[0:(NROW - NBLK * RBH), :] = tail_ref[...]


@jax.jit
def _tc_pack(WT, tail):
    hi_max = (V - RBH) // RBH
    out = pl.pallas_call(
        _tpose_body,
        grid=(NBLK + 1,),
        in_specs=[pl.BlockSpec((D, RBH), lambda i: (0, jnp.minimum(i, NBLK - 1))),
                  pl.BlockSpec((D, RBH),
                               lambda i: (0, jnp.minimum(i + NBLK, hi_max))),
                  pl.BlockSpec((NROW - NBLK * RBH, 2 * D), lambda i: (0, 0))],
        out_specs=pl.BlockSpec((RBH, 2 * D), lambda i: (i, 0)),
        out_shape=jax.ShapeDtypeStruct((NROW, 2 * D), jnp.float32),
    )(WT, WT, tail)
    return jnp.reshape(out, (V, D))


def _pack_table(W):
    tail = jnp.reshape(W[VTAIL:, :], (NROW - NBLK * RBH, 2 * D))
    return _tc_pack(W.T, tail)


def _lin_idx(idx):
    idx = idx.astype(jnp.int32)
    return jnp.where(idx < VPAD, 2 * idx,
                     jnp.where(idx < VTAIL, 2 * (idx - VPAD) + 1, idx))


def kernel(center, contexts, negatives, W_center, W_context):
    ctxT = _lin_idx(contexts.T)
    cnF = _lin_idx(jnp.concatenate(
        [center[:, None], negatives], axis=1)).reshape(B * K)
    scores = _sc_scores(_pack_table(W_center), _pack_table(W_context),
                        ctxT, cnF)
    return _tc_loss(scores)[0, 0]


# speedup vs baseline: 1.1409x; 1.1409x over previous
import dataclasses
import functools

import jax
import jax.numpy as jnp
from jax import lax
from jax.experimental import pallas as pl
from jax.experimental.pallas import tpu as pltpu
from jax.experimental.pallas import tpu_sc as plsc

V = 1000000
D = 64
B = 16384
C = 20
K = 17

NW = 32
PER_W = B // NW
CHUNK = 32
NCHUNK = PER_W // CHUNK
NB = 1


def _sc_compiler_params():
    cp = pltpu.CompilerParams(use_tc_tiling_on_sc=False)
    if "needs_layout_passes" in pltpu.CompilerParams.__dataclass_fields__:
        cp = dataclasses.replace(cp, needs_layout_passes=False)
    return cp


CHUNK_CTX = 128
NCH_CTX = PER_W // CHUNK_CTX


def _sc_ctx_body(wctx, ctxT, cv_out, ctxi, accA, accB, semA, semB):
    wid = lax.axis_index("s") * 2 + lax.axis_index("c")
    wbase = wid * PER_W
    pltpu.sync_copy(ctxT.at[:, pl.ds(wbase, PER_W)], ctxi)
    zeros16 = jnp.zeros((16,), jnp.float32)

    def zero_acc(acc):
        @pl.loop(0, CHUNK_CTX)
        def _(i):
            for j in range(D // 16):
                acc[i, pl.ds(j * 16, 16)] = zeros16

    def issue(n, acc, sem):
        for c in range(C):
            pltpu.async_copy(wctx.at[ctxi.at[c, pl.ds(n * CHUNK_CTX,
                                                      CHUNK_CTX)]],
                             acc, sem, add=True)

    def wait(acc, sem):
        for c in range(C):
            pltpu.make_async_copy(
                wctx.at[ctxi.at[0, pl.ds(0, CHUNK_CTX)]], acc, sem).wait()

    def do_chunk(n, acc_cur, sem_cur, acc_nxt, sem_nxt):
        @pl.when(n + 1 < NCH_CTX)
        def _():
            zero_acc(acc_nxt)
            issue(n + 1, acc_nxt, sem_nxt)

        wait(acc_cur, sem_cur)
        pltpu.sync_copy(acc_cur,
                        cv_out.at[pl.ds(wbase + n * CHUNK_CTX, CHUNK_CTX), :])

    zero_acc(accA)
    issue(0, accA, semA)

    @pl.loop(0, NCH_CTX, step=2)
    def _(s):
        do_chunk(s, accA, semA, accB, semB)
        do_chunk(s + 1, accB, semB, accA, semA)


@jax.jit
def _sc_ctx(wctx, ctxT):
    mesh = plsc.VectorSubcoreMesh(core_axis_name="c", subcore_axis_name="s")
    f = pl.kernel(
        _sc_ctx_body,
        out_type=jax.ShapeDtypeStruct((B, D), jnp.float32),
        mesh=mesh,
        scratch_types=[
            pltpu.VMEM((C, PER_W), jnp.int32),
            pltpu.VMEM((CHUNK_CTX, D), jnp.float32),
            pltpu.VMEM((CHUNK_CTX, D), jnp.float32),
            pltpu.SemaphoreType.DMA,
            pltpu.SemaphoreType.DMA,
        ],
        compiler_params=_sc_compiler_params(),
    )
    return f(wctx, ctxT)


def _sc_dots_body(wcen, cnF, cvh, out, cni, cvv, scores, rowsA, rowsB,
                  semA, semB):
    wid = lax.axis_index("s") * 2 + lax.axis_index("c")
    wbase = wid * PER_W
    pltpu.sync_copy(cnF.at[pl.ds(wbase * K, PER_W * K)], cni)
    pltpu.sync_copy(cvh.at[pl.ds(wbase, PER_W), :], cvv)

    def issue(n, rows, sem):
        pltpu.async_copy(wcen.at[cni.at[pl.ds(n * CHUNK * K, CHUNK * K)]],
                         rows, sem)

    def wait(rows, sem):
        pltpu.make_async_copy(
            wcen.at[cni.at[pl.ds(0, CHUNK * K)]], rows, sem).wait()

    def do_chunk(n, rows_cur, sem_cur, rows_nxt, sem_nxt):
        @pl.when(n + 1 < NCHUNK)
        def _():
            issue(n + 1, rows_nxt, sem_nxt)

        wait(rows_cur, sem_cur)

        base = n * CHUNK
        lane = lax.iota(jnp.int32, 16)
        last = lane == 15

        @pl.loop(0, CHUNK)
        def _(i):
            cv = [cvv[base + i, pl.ds(j * 16, 16)] for j in range(D // 16)]
            col = jnp.full((16,), base + i, jnp.int32)
            for k in range(K):
                w = rows_cur[i * K + k, pl.ds(0, 16)] * cv[0]
                for j in range(1, D // 16):
                    w = w + rows_cur[i * K + k, pl.ds(j * 16, 16)] * cv[j]
                plsc.store_scatter(scores,
                                   [jnp.full((16,), k, jnp.int32), col],
                                   jnp.cumsum(w), mask=last)

    issue(0, rowsA, semA)

    @pl.loop(0, NCHUNK, step=2)
    def _(s):
        do_chunk(s, rowsA, semA, rowsB, semB)
        do_chunk(s + 1, rowsB, semB, rowsA, semA)

    pltpu.sync_copy(scores, out.at[:, pl.ds(wbase, PER_W)])


@jax.jit
def _sc_dots(wcen, cnF, cv):
    mesh = plsc.VectorSubcoreMesh(core_axis_name="c", subcore_axis_name="s")
    f = pl.kernel(
        _sc_dots_body,
        out_type=jax.ShapeDtypeStruct((K, B), jnp.float32),
        mesh=mesh,
        scratch_types=[
            pltpu.VMEM((PER_W * K,), jnp.int32),
            pltpu.VMEM((PER_W, D), jnp.float32),
            pltpu.VMEM((K, PER_W), jnp.float32),
            pltpu.VMEM((CHUNK * K, D), jnp.float32),
            pltpu.VMEM((CHUNK * K, D), jnp.float32),
            pltpu.SemaphoreType.DMA,
            pltpu.SemaphoreType.DMA,
        ],
        compiler_params=_sc_compiler_params(),
    )
    return f(wcen, cnF, cv)


def _tc_loss_body(s_ref, o_ref):
    x = s_ref[...]
    zp = x[0:1, :]
    zn = -x[1:K, :]
    lsp = jnp.minimum(zp, 0.0) - jnp.log1p(jnp.exp(-jnp.abs(zp)))
    lsn = jnp.minimum(zn, 0.0) - jnp.log1p(jnp.exp(-jnp.abs(zn)))
    o_ref[0, 0] = -(jnp.sum(lsp) + jnp.sum(lsn)) / float(K * B)


@jax.jit
def _tc_loss(scores):
    return pl.pallas_call(
        _tc_loss_body,
        out_shape=jax.ShapeDtypeStruct((1, 1), jnp.float32),
        in_specs=[pl.BlockSpec((K, B), lambda: (0, 0))],
        out_specs=pl.BlockSpec(memory_space=pltpu.SMEM),
    )(scores)


RBH = 8192
NBLK = 61
VPAD = NBLK * RBH
VTAIL = 2 * VPAD
NROW = V // 2


def _tpose_body(lo_ref, hi_ref, tail_ref, out_ref):
    i = pl.program_id(0)
    eye = (lax.broadcasted_iota(jnp.int32, (2 * D, 2 * D), 0) ==
           lax.broadcasted_iota(jnp.int32, (2 * D, 2 * D), 1)).astype(jnp.float32)

    @pl.when(i < NBLK)
    def _():
        x = jnp.concatenate([lo_ref[...], hi_ref[...]], axis=0)
        out_ref[...] = lax.dot_general(x, eye, (((0,), (0,)), ((), ())),
                                       preferred_element_type=jnp.float32)

    @pl.when(i == NBLK)
    def _():
        out_ref[0:(NROW - NBLK * RBH), :] = tail_ref[...]


@jax.jit
def _tc_pack(WT, tail):
    hi_max = (V - RBH) // RBH
    out = pl.pallas_call(
        _tpose_body,
        grid=(NBLK + 1,),
        in_specs=[pl.BlockSpec((D, RBH), lambda i: (0, jnp.minimum(i, NBLK - 1))),
                  pl.BlockSpec((D, RBH),
                               lambda i: (0, jnp.minimum(i + NBLK, hi_max))),
                  pl.BlockSpec((NROW - NBLK * RBH, 2 * D), lambda i: (0, 0))],
        out_specs=pl.BlockSpec((RBH, 2 * D), lambda i: (i, 0)),
        out_shape=jax.ShapeDtypeStruct((NROW, 2 * D), jnp.float32),
    )(WT, WT, tail)
    return jnp.reshape(out, (V, D))


def _pack_table(W):
    tail = jnp.reshape(W[VTAIL:, :], (NROW - NBLK * RBH, 2 * D))
    return _tc_pack(W.T, tail)


def _lin_idx(idx):
    idx = idx.astype(jnp.int32)
    return jnp.where(idx < VPAD, 2 * idx,
                     jnp.where(idx < VTAIL, 2 * (idx - VPAD) + 1, idx))


def kernel(center, contexts, negatives, W_center, W_context):
    ctxT = _lin_idx(contexts.T)
    cnF = _lin_idx(jnp.concatenate(
        [center[:, None], negatives], axis=1)).reshape(B * K)
    cv = _sc_ctx(_pack_table(W_context), ctxT)
    scores = _sc_dots(_pack_table(W_center), cnF, cv)
    return _tc_loss(scores)[0, 0]


# speedup vs baseline: 1.1799x; 1.0342x over previous
import dataclasses
import functools

import jax
import jax.numpy as jnp
from jax import lax
from jax.experimental import pallas as pl
from jax.experimental.pallas import tpu as pltpu
from jax.experimental.pallas import tpu_sc as plsc

V = 1000000
D = 64
B = 16384
C = 20
K = 17

NW = 32
PER_W = B // NW
CHUNK = 32
NCHUNK = PER_W // CHUNK
NB = 1


def _sc_scores_body(wcen, wctx, ctxT, cnF, out, ctxi, cni, scores,
                    accA, accB, rowsA, rowsB,
                    sem_cA, sem_cB, sem_rA, sem_rB):
    wid = lax.axis_index("s") * 2 + lax.axis_index("c")
    wbase = wid * PER_W

    pltpu.sync_copy(ctxT.at[:, pl.ds(wbase, PER_W)], ctxi)
    pltpu.sync_copy(cnF.at[pl.ds(wbase * K, PER_W * K)], cni)

    zeros16 = jnp.zeros((16,), jnp.float32)

    def zero_acc(acc):
        @pl.loop(0, CHUNK)
        def _(i):
            for b in range(NB):
                for j in range(D // 16):
                    acc[b, i, pl.ds(j * 16, 16)] = zeros16

    def issue(n, acc, rows, sem_c, sem_r):
        for c in range(C):
            pltpu.async_copy(wctx.at[ctxi.at[c, pl.ds(n * CHUNK, CHUNK)]],
                             acc.at[c % NB], sem_c, add=True)
        pltpu.async_copy(wcen.at[cni.at[pl.ds(n * CHUNK * K, CHUNK * K)]],
                         rows, sem_r)

    def wait(acc, rows, sem_c, sem_r):
        for c in range(C):
            pltpu.make_async_copy(
                wctx.at[ctxi.at[0, pl.ds(0, CHUNK)]], acc.at[c % NB],
                sem_c).wait()
        pltpu.make_async_copy(
            wcen.at[cni.at[pl.ds(0, CHUNK * K)]], rows, sem_r).wait()

    def do_chunk(n, acc_cur, rows_cur, sem_c_cur, sem_r_cur,
                 acc_nxt, rows_nxt, sem_c_nxt, sem_r_nxt):
        @pl.when(n + 1 < NCHUNK)
        def _():
            zero_acc(acc_nxt)
            issue(n + 1, acc_nxt, rows_nxt, sem_c_nxt, sem_r_nxt)

        wait(acc_cur, rows_cur, sem_c_cur, sem_r_cur)

        base = n * CHUNK
        lane = lax.iota(jnp.int32, 16)
        last = lane == 15

        def item(i):
            cv = [acc_cur[0, i, pl.ds(j * 16, 16)] for j in range(D // 16)]
            for b in range(1, NB):
                cv = [cv[j] + acc_cur[b, i, pl.ds(j * 16, 16)]
                      for j in range(D // 16)]
            col = jnp.full((16,), base + i, jnp.int32)
            for k in range(K):
                w = rows_cur[i * K + k, pl.ds(0, 16)] * cv[0]
                for j in range(1, D // 16):
                    w = w + rows_cur[i * K + k, pl.ds(j * 16, 16)] * cv[j]
                plsc.store_scatter(scores,
                                   [jnp.full((16,), k, jnp.int32), col],
                                   jnp.cumsum(w), mask=last)

        @pl.loop(0, CHUNK, step=2)
        def _(i):
            item(i)
            item(i + 1)

    zero_acc(accA)
    issue(0, accA, rowsA, sem_cA, sem_rA)

    @pl.loop(0, NCHUNK, step=2)
    def _(s):
        do_chunk(s, accA, rowsA, sem_cA, sem_rA, accB, rowsB, sem_cB, sem_rB)
        do_chunk(s + 1, accB, rowsB, sem_cB, sem_rB, accA, rowsA, sem_cA, sem_rA)

    pltpu.sync_copy(scores, out.at[:, pl.ds(wbase, PER_W)])


@jax.jit
def _sc_scores(wcen, wctx, ctxT, cnT):
    mesh = plsc.VectorSubcoreMesh(core_axis_name="c", subcore_axis_name="s")
    cp = pltpu.CompilerParams(use_tc_tiling_on_sc=False)
    if "needs_layout_passes" in pltpu.CompilerParams.__dataclass_fields__:
        cp = dataclasses.replace(cp, needs_layout_passes=False)
    f = pl.kernel(
        _sc_scores_body,
        out_type=jax.ShapeDtypeStruct((K, B), jnp.float32),
        mesh=mesh,
        scratch_types=[
            pltpu.VMEM((C, PER_W), jnp.int32),
            pltpu.VMEM((PER_W * K,), jnp.int32),
            pltpu.VMEM((K, PER_W), jnp.float32),
            pltpu.VMEM((NB, CHUNK, D), jnp.float32),
            pltpu.VMEM((NB, CHUNK, D), jnp.float32),
            pltpu.VMEM((CHUNK * K, D), jnp.float32),
            pltpu.VMEM((CHUNK * K, D), jnp.float32),
            pltpu.SemaphoreType.DMA,
            pltpu.SemaphoreType.DMA,
            pltpu.SemaphoreType.DMA,
            pltpu.SemaphoreType.DMA,
        ],
        compiler_params=cp,
    )
    return f(wcen, wctx, ctxT, cnT)


def _tc_loss_body(s_ref, o_ref):
    x = s_ref[...]
    zp = x[0:1, :]
    zn = -x[1:K, :]
    lsp = jnp.minimum(zp, 0.0) - jnp.log1p(jnp.exp(-jnp.abs(zp)))
    lsn = jnp.minimum(zn, 0.0) - jnp.log1p(jnp.exp(-jnp.abs(zn)))
    o_ref[0, 0] = -(jnp.sum(lsp) + jnp.sum(lsn)) / float(K * B)


@jax.jit
def _tc_loss(scores):
    return pl.pallas_call(
        _tc_loss_body,
        out_shape=jax.ShapeDtypeStruct((1, 1), jnp.float32),
        in_specs=[pl.BlockSpec((K, B), lambda: (0, 0))],
        out_specs=pl.BlockSpec(memory_space=pltpu.SMEM),
    )(scores)


RBH = 8192
NBLK = 61
VPAD = NBLK * RBH
VTAIL = 2 * VPAD
NROW = V // 2


def _tpose_body(lo_ref, hi_ref, tail_ref, out_ref):
    i = pl.program_id(0)
    eye = (lax.broadcasted_iota(jnp.int32, (2 * D, 2 * D), 0) ==
           lax.broadcasted_iota(jnp.int32, (2 * D, 2 * D), 1)).astype(jnp.float32)

    @pl.when(i < NBLK)
    def _():
        x = jnp.concatenate([lo_ref[...], hi_ref[...]], axis=0)
        out_ref[...] = lax.dot_general(x, eye, (((0,), (0,)), ((), ())),
                                       preferred_element_type=jnp.float32)

    @pl.when(i == NBLK)
    def _():
        out_ref[0:(NROW - NBLK * RBH), :] = tail_ref[...]


@jax.jit
def _tc_pack(WT, tail):
    hi_max = (V - RBH) // RBH
    out = pl.pallas_call(
        _tpose_body,
        grid=(NBLK + 1,),
        in_specs=[pl.BlockSpec((D, RBH), lambda i: (0, jnp.minimum(i, NBLK - 1))),
                  pl.BlockSpec((D, RBH),
                               lambda i: (0, jnp.minimum(i + NBLK, hi_max))),
                  pl.BlockSpec((NROW - NBLK * RBH, 2 * D), lambda i: (0, 0))],
        out_specs=pl.BlockSpec((RBH, 2 * D), lambda i: (i, 0)),
        out_shape=jax.ShapeDtypeStruct((NROW, 2 * D), jnp.float32),
    )(WT, WT, tail)
    return jnp.reshape(out, (V, D))


def _pack_table(W):
    tail = jnp.reshape(W[VTAIL:, :], (NROW - NBLK * RBH, 2 * D))
    return _tc_pack(W.T, tail)


def _lin_idx(idx):
    idx = idx.astype(jnp.int32)
    return jnp.where(idx < VPAD, 2 * idx,
                     jnp.where(idx < VTAIL, 2 * (idx - VPAD) + 1, idx))


def kernel(center, contexts, negatives, W_center, W_context):
    ctxT = _lin_idx(contexts.T)
    cnF = _lin_idx(jnp.concatenate(
        [center[:, None], negatives], axis=1)).reshape(B * K)
    scores = _sc_scores(_pack_table(W_center), _pack_table(W_context),
                        ctxT, cnF)
    return _tc_loss(scores)[0, 0]


# speedup vs baseline: 1.1829x; 1.0025x over previous
import dataclasses

import jax
import jax.numpy as jnp
from jax import lax
from jax.experimental import pallas as pl
from jax.experimental.pallas import tpu as pltpu
from jax.experimental.pallas import tpu_sc as plsc

V = 1000000
D = 64
B = 16384
C = 20
K = 17

NW = 32
PER_W = B // NW
CHUNK = 32
NCHUNK = PER_W // CHUNK
NB = 1


def _sc_scores_body(wcen, wctx, ctxT, cnF, out, ctxi, cni, scores,
                    accA, accB, rowsA, rowsB,
                    sem_cA, sem_cB, sem_rA, sem_rB):
    wid = lax.axis_index("s") * 2 + lax.axis_index("c")
    wbase = wid * PER_W

    pltpu.sync_copy(ctxT.at[:, pl.ds(wbase, PER_W)], ctxi)
    pltpu.sync_copy(cnF.at[pl.ds(wbase * K, PER_W * K)], cni)

    zeros16 = jnp.zeros((16,), jnp.float32)

    def zero_acc(acc):
        @pl.loop(0, CHUNK)
        def _(i):
            for b in range(NB):
                for j in range(D // 16):
                    acc[b, i, pl.ds(j * 16, 16)] = zeros16

    def issue(n, acc, rows, sem_c, sem_r):
        for c in range(C):
            pltpu.async_copy(wctx.at[ctxi.at[c, pl.ds(n * CHUNK, CHUNK)]],
                             acc.at[c % NB], sem_c, add=True)
        pltpu.async_copy(wcen.at[cni.at[pl.ds(n * CHUNK * K, CHUNK * K)]],
                         rows, sem_r)

    def wait(acc, rows, sem_c, sem_r):
        for c in range(C):
            pltpu.make_async_copy(
                wctx.at[ctxi.at[0, pl.ds(0, CHUNK)]], acc.at[c % NB],
                sem_c).wait()
        pltpu.make_async_copy(
            wcen.at[cni.at[pl.ds(0, CHUNK * K)]], rows, sem_r).wait()

    def do_chunk(n, acc_cur, rows_cur, sem_c_cur, sem_r_cur,
                 acc_nxt, rows_nxt, sem_c_nxt, sem_r_nxt):
        @pl.when(n + 1 < NCHUNK)
        def _():
            zero_acc(acc_nxt)
            issue(n + 1, acc_nxt, rows_nxt, sem_c_nxt, sem_r_nxt)

        wait(acc_cur, rows_cur, sem_c_cur, sem_r_cur)

        base = n * CHUNK
        lane = lax.iota(jnp.int32, 16)
        last = lane == 15

        @pl.loop(0, CHUNK)
        def _(i):
            cv = [acc_cur[0, i, pl.ds(j * 16, 16)] for j in range(D // 16)]
            for b in range(1, NB):
                cv = [cv[j] + acc_cur[b, i, pl.ds(j * 16, 16)]
                      for j in range(D // 16)]
            col = jnp.full((16,), base + i, jnp.int32)
            for k in range(K):
                w = rows_cur[i * K + k, pl.ds(0, 16)] * cv[0]
                for j in range(1, D // 16):
                    w = w + rows_cur[i * K + k, pl.ds(j * 16, 16)] * cv[j]
                plsc.store_scatter(scores,
                                   [jnp.full((16,), k, jnp.int32), col],
                                   jnp.cumsum(w), mask=last)

    zero_acc(accA)
    issue(0, accA, rowsA, sem_cA, sem_rA)

    @pl.loop(0, NCHUNK, step=2)
    def _(s):
        do_chunk(s, accA, rowsA, sem_cA, sem_rA, accB, rowsB, sem_cB, sem_rB)
        do_chunk(s + 1, accB, rowsB, sem_cB, sem_rB, accA, rowsA, sem_cA, sem_rA)

    pltpu.sync_copy(scores, out.at[:, pl.ds(wbase, PER_W)])


@jax.jit
def _sc_scores(wcen, wctx, ctxT, cnT):
    mesh = plsc.VectorSubcoreMesh(core_axis_name="c", subcore_axis_name="s")
    cp = pltpu.CompilerParams(use_tc_tiling_on_sc=False)
    if "needs_layout_passes" in pltpu.CompilerParams.__dataclass_fields__:
        cp = dataclasses.replace(cp, needs_layout_passes=False)
    f = pl.kernel(
        _sc_scores_body,
        out_type=jax.ShapeDtypeStruct((K, B), jnp.float32),
        mesh=mesh,
        scratch_types=[
            pltpu.VMEM((C, PER_W), jnp.int32),
            pltpu.VMEM((PER_W * K,), jnp.int32),
            pltpu.VMEM((K, PER_W), jnp.float32),
            pltpu.VMEM((NB, CHUNK, D), jnp.float32),
            pltpu.VMEM((NB, CHUNK, D), jnp.float32),
            pltpu.VMEM((CHUNK * K, D), jnp.float32),
            pltpu.VMEM((CHUNK * K, D), jnp.float32),
            pltpu.SemaphoreType.DMA,
            pltpu.SemaphoreType.DMA,
            pltpu.SemaphoreType.DMA,
            pltpu.SemaphoreType.DMA,
        ],
        compiler_params=cp,
    )
    return f(wcen, wctx, ctxT, cnT)


def _tc_loss_body(s_ref, o_ref):
    x = s_ref[...]
    zp = x[0:1, :]
    zn = -x[1:K, :]
    lsp = jnp.minimum(zp, 0.0) - jnp.log1p(jnp.exp(-jnp.abs(zp)))
    lsn = jnp.minimum(zn, 0.0) - jnp.log1p(jnp.exp(-jnp.abs(zn)))
    o_ref[0, 0] = -(jnp.sum(lsp) + jnp.sum(lsn)) / float(K * B)


@jax.jit
def _tc_loss(scores):
    return pl.pallas_call(
        _tc_loss_body,
        out_shape=jax.ShapeDtypeStruct((1, 1), jnp.float32),
        in_specs=[pl.BlockSpec((K, B), lambda: (0, 0))],
        out_specs=pl.BlockSpec(memory_space=pltpu.SMEM),
    )(scores)


RBH = 8192
NBLK = 61
VPAD = NBLK * RBH
VTAIL = 2 * VPAD
NROW = V // 2


def _tpose_body(lo_ref, hi_ref, tail_ref, out_ref):
    i = pl.program_id(0)
    eye = (lax.broadcasted_iota(jnp.int32, (2 * D, 2 * D), 0) ==
           lax.broadcasted_iota(jnp.int32, (2 * D, 2 * D), 1)).astype(jnp.float32)

    @pl.when(i < NBLK)
    def _():
        x = jnp.concatenate([lo_ref[...], hi_ref[...]], axis=0)
        out_ref[...] = lax.dot_general(x, eye, (((0,), (0,)), ((), ())),
                                       preferred_element_type=jnp.float32)

    @pl.when(i == NBLK)
    def _():
        out_ref[0:(NROW - NBLK * RBH), :] = tail_ref[...]


@jax.jit
def _tc_pack(WT, tail):
    hi_max = (V - RBH) // RBH
    out = pl.pallas_call(
        _tpose_body,
        grid=(NBLK + 1,),
        in_specs=[pl.BlockSpec((D, RBH), lambda i: (0, jnp.minimum(i, NBLK - 1))),
                  pl.BlockSpec((D, RBH),
                               lambda i: (0, jnp.minimum(i + NBLK, hi_max))),
                  pl.BlockSpec((NROW - NBLK * RBH, 2 * D), lambda i: (0, 0))],
        out_specs=pl.BlockSpec((RBH, 2 * D), lambda i: (i, 0)),
        out_shape=jax.ShapeDtypeStruct((NROW, 2 * D), jnp.float32),
    )(WT, WT, tail)
    return jnp.reshape(out, (V, D))


def _pack_table(W):
    tail = jnp.reshape(W[VTAIL:, :], (NROW - NBLK * RBH, 2 * D))
    return _tc_pack(W.T, tail)


def _lin_idx(idx):
    idx = idx.astype(jnp.int32)
    return jnp.where(idx < VPAD, 2 * idx,
                     jnp.where(idx < VTAIL, 2 * (idx - VPAD) + 1, idx))


def kernel(center, contexts, negatives, W_center, W_context):
    ctxT = _lin_idx(contexts.T)
    cnF = _lin_idx(jnp.concatenate(
        [center[:, None], negatives], axis=1)).reshape(B * K)
    scores = _sc_scores(_pack_table(W_center), _pack_table(W_context),
                        ctxT, cnF)
    return _tc_loss(scores)[0, 0]
